# baseline (speedup 1.0000x reference)
import jax
import jax.numpy as jnp
from jax.experimental import pallas as pl


def _sigmoid_body(x_ref, o_ref):
    o_ref[...] = jax.nn.sigmoid(x_ref[...])


def kernel(pred_logits, pred_params, target_sizes):
    B, N, C = pred_logits.shape
    num_select = 100
    flat = pred_logits.reshape(B, 1, N * C)
    prob = pl.pallas_call(
        _sigmoid_body,
        grid=(B,),
        in_specs=[pl.BlockSpec((1, 1, N * C), lambda i: (i, 0, 0))],
        out_specs=pl.BlockSpec((1, 1, N * C), lambda i: (i, 0, 0)),
        out_shape=jax.ShapeDtypeStruct((B, 1, N * C), jnp.float32),
    )(flat).reshape(B, N * C)
    topk_values, topk_indexes = jax.lax.top_k(prob, num_select)
    scores = topk_values
    topk_boxes = topk_indexes // C
    labels = topk_indexes % C
    boxes = pred_params
    idx = jnp.broadcast_to(topk_boxes[:, :, None], (B, num_select, boxes.shape[2]))
    boxes = jnp.take_along_axis(boxes, idx, axis=1)
    img_h = target_sizes[:, 0].astype(jnp.float32)
    img_w = target_sizes[:, 1].astype(jnp.float32)
    scale_fct = jnp.stack([img_w, img_h] * 9, axis=1)
    boxes = boxes * scale_fct[:, None, :]
    return scores, labels, boxes


# speedup vs baseline: 5.4237x; 5.4237x over previous
import jax
import jax.numpy as jnp
from jax import lax
from jax.experimental import pallas as pl
from jax.experimental.pallas import tpu as pltpu
from jax.experimental.pallas import tpu_sc as plsc

_B = 32
_N = 8192
_C = 91
_NC = _N * _C
_K = 100
_WIN = 8192
_NWIN = _NC // _WIN
_HB = 15
_NBUCKET = 1 << _HB
_SHIFT = 32 - _HB
_CAP = 1024
_GPAD = 128
_GD = 24
_MININT = -(2 ** 31)

_sc_mesh = plsc.VectorSubcoreMesh(core_axis_name="c", subcore_axis_name="s")


def _worker_id():
    return lax.axis_index("s") * 2 + lax.axis_index("c")


def _bucketize(x):
    bits = lax.bitcast_convert_type(x, jnp.int32)
    m = lax.shift_right_arithmetic(bits, 31)
    key_u = bits ^ (m | _MININT)
    return lax.shift_right_logical(key_u, _SHIFT)


def _select_body(logits_hbm, vals_hbm, idx_hbm,
                 hist, wbuf, cvals, cidx, wmax, sem0, sem1):
    wid = _worker_id()
    zer = jnp.zeros((16,), jnp.int32)
    ones16 = jnp.ones((16,), jnp.int32)
    neg = jnp.full((16,), -jnp.inf, jnp.float32)
    iota16 = lax.iota(jnp.int32, 16)

    def zi(i, _):
        hist[pl.ds(i * 16, 16)] = zer
        return 0
    lax.fori_loop(0, _NBUCKET // 16, zi, 0)

    def zc(i, _):
        cvals[pl.ds(i * 16, 16)] = neg
        cidx[pl.ds(i * 16, 16)] = zer
        return 0
    lax.fori_loop(0, _CAP // 16, zc, 0)

    def dma(w, b, sem):
        return pltpu.make_async_copy(
            logits_hbm.at[wid, pl.ds(w * _WIN, _WIN)], wbuf.at[b], sem)

    def histo_window(w, b):
        def inner(i, mx):
            b0 = _bucketize(wbuf[b, pl.ds(i * 64, 16)])
            b1 = _bucketize(wbuf[b, pl.ds(i * 64 + 16, 16)])
            b2 = _bucketize(wbuf[b, pl.ds(i * 64 + 32, 16)])
            b3 = _bucketize(wbuf[b, pl.ds(i * 64 + 48, 16)])
            plsc.addupdate_scatter(hist, [b0], ones16)
            plsc.addupdate_scatter(hist, [b1], ones16)
            plsc.addupdate_scatter(hist, [b2], ones16)
            plsc.addupdate_scatter(hist, [b3], ones16)
            return jnp.maximum(jnp.maximum(mx, jnp.maximum(b0, b1)),
                               jnp.maximum(b2, b3))
        mx = lax.fori_loop(0, _WIN // 64, inner, zer)
        wmax[w] = jnp.max(mx)
        return mx

    dma(0, 0, sem0).start()
    dma(1, 1, sem1).start()

    def pair(g, gmx):
        w0 = 2 * g
        dma(w0, 0, sem0).wait()
        mx0 = histo_window(w0, 0)
        dma(w0 + 2, 0, sem0).start()
        dma(w0 + 1, 1, sem1).wait()
        mx1 = histo_window(w0 + 1, 1)

        @pl.when(w0 + 3 < _NWIN)
        def _():
            dma(w0 + 3, 1, sem1).start()
        return jnp.maximum(gmx, jnp.maximum(mx0, mx1))

    gmx = lax.fori_loop(0, (_NWIN - 1) // 2, pair, zer)
    dma(_NWIN - 1, 0, sem0).wait()
    gmx = jnp.maximum(gmx, histo_window(_NWIN - 1, 0))
    bmax = jnp.max(gmx)

    def cond(c):
        return c[2] < 0

    def wb(c):
        j, cum, bst = c
        v = hist[pl.ds(j * 16, 16)]
        rv = lax.rev(v, (0,))
        cs = jnp.cumsum(rv)
        nhit = jnp.sum((cs + cum >= _K).astype(jnp.int32))
        bst2 = jnp.where(nhit > 0, j * 16 + nhit - 1, bst)
        return (j - 1, cum + jnp.sum(v), bst2)

    _, _, bstar = lax.while_loop(
        cond, wb, (bmax >> 4, jnp.int32(0), jnp.int32(-1)))
    beff = jnp.maximum(bstar - 1, 0)

    def cwin(w, off):
        def do(o):
            pltpu.sync_copy(logits_hbm.at[wid, pl.ds(w * _WIN, _WIN)],
                            wbuf.at[0])
            base = w * _WIN

            def inner(i, o2):
                x = wbuf[0, pl.ds(i * 16, 16)]
                msk = _bucketize(x) >= beff
                oc = jnp.minimum(o2, _CAP - 16)
                plsc.store_compressed(cvals.at[pl.ds(oc, 16)], x, mask=msk)
                plsc.store_compressed(cidx.at[pl.ds(oc, 16)],
                                      iota16 + (base + i * 16), mask=msk)
                return o2 + jnp.sum(msk.astype(jnp.int32))
            return lax.fori_loop(0, _WIN // 16, inner, o)
        return lax.cond(wmax[w] >= beff, do, lambda o: o, off)

    lax.fori_loop(0, _NWIN, cwin, jnp.int32(0))

    pltpu.sync_copy(cvals, vals_hbm.at[wid])
    pltpu.sync_copy(cidx, idx_hbm.at[wid])


_select_candidates = pl.kernel(
    _select_body,
    out_type=[jax.ShapeDtypeStruct((_B, _CAP), jnp.float32),
              jax.ShapeDtypeStruct((_B, _CAP), jnp.int32)],
    mesh=_sc_mesh,
    compiler_params=pltpu.CompilerParams(needs_layout_passes=False),
    scratch_types=[
        pltpu.VMEM((_NBUCKET,), jnp.int32),
        pltpu.VMEM((2, _WIN), jnp.float32),
        pltpu.VMEM((_CAP,), jnp.float32),
        pltpu.VMEM((_CAP,), jnp.int32),
        pltpu.SMEM((_NWIN + 1,), jnp.int32),
        pltpu.SemaphoreType.DMA,
        pltpu.SemaphoreType.DMA,
    ],
)


def _gather_body(params_hbm, rows_hbm, out_hbm, idxv, rowsv, sem):
    wid = _worker_id()
    pltpu.sync_copy(rows_hbm.at[wid], idxv)
    pltpu.async_copy(params_hbm.at[idxv], rowsv, sem).wait()
    pltpu.sync_copy(rowsv, out_hbm.at[wid])


_gather_rows = pl.kernel(
    _gather_body,
    out_type=jax.ShapeDtypeStruct((_B, _GPAD, _GD), jnp.float32),
    mesh=_sc_mesh,
    compiler_params=pltpu.CompilerParams(
        needs_layout_passes=False, use_tc_tiling_on_sc=False),
    scratch_types=[
        pltpu.VMEM((_GPAD,), jnp.int32),
        pltpu.VMEM((_GPAD, _GD), jnp.float32),
        pltpu.SemaphoreType.DMA,
    ],
)


def kernel(pred_logits, pred_params, target_sizes):
    B, N, C = pred_logits.shape
    flat = pred_logits.reshape(B, N * C)
    cand_vals, cand_idx = _select_candidates(flat)
    p = jax.nn.sigmoid(cand_vals)
    scores, pos = lax.top_k(p, _K)
    flat_idx = jnp.take_along_axis(cand_idx, pos, axis=1)
    labels = flat_idx % C
    rowbase = (jnp.arange(B, dtype=jnp.int32) * N)[:, None]
    rows = flat_idx // C + rowbase
    pad = jnp.broadcast_to(rowbase, (B, _GPAD - _K))
    rows_pad = jnp.concatenate([rows, pad], axis=1).astype(jnp.int32)
    params_pad = jnp.pad(pred_params.reshape(B * N, -1),
                         ((0, 0), (0, _GD - pred_params.shape[2])))
    boxes_g = _gather_rows(params_pad, rows_pad)
    img_h = target_sizes[:, 0].astype(jnp.float32)
    img_w = target_sizes[:, 1].astype(jnp.float32)
    scale_fct = jnp.stack([img_w, img_h] * 9, axis=1)
    boxes = boxes_g[:, :_K, :pred_params.shape[2]] * scale_fct[:, None, :]
    return scores, labels, boxes


# speedup vs baseline: 8.4487x; 1.5577x over previous
import jax
import jax.numpy as jnp
from jax import lax
from jax.experimental import pallas as pl
from jax.experimental.pallas import tpu as pltpu
from jax.experimental.pallas import tpu_sc as plsc

_B = 32
_N = 8192
_C = 91
_NC = _N * _C
_K = 100
_WIN = 8192
_NWIN = _NC // _WIN
_HB = 15
_NBUCKET = 1 << _HB
_SHIFT = 32 - _HB
_CAP = 512
_SEGV = 64
_NSEG = _WIN // (_SEGV * 16)
_GPAD = 128
_GD = 24
_MININT = -(2 ** 31)

_sc_mesh = plsc.VectorSubcoreMesh(core_axis_name="c", subcore_axis_name="s")


def _worker_id():
    return lax.axis_index("s") * 2 + lax.axis_index("c")


def _bucketize(x):
    bits = lax.bitcast_convert_type(x, jnp.int32)
    m = lax.shift_right_arithmetic(bits, 31)
    key_u = bits ^ (m | _MININT)
    return lax.shift_right_logical(key_u, _SHIFT)


def _select_body(logits_hbm, vals_hbm, idx_hbm,
                 hist, wbuf, cvals, cidx, wmaxb, segb, sem0, sem1):
    wid = _worker_id()
    zer = jnp.zeros((16,), jnp.int32)
    ones16 = jnp.ones((16,), jnp.int32)
    neg = jnp.full((16,), -jnp.inf, jnp.float32)
    iota16 = lax.iota(jnp.int32, 16)

    def zi(i, _):
        hist[pl.ds(i * 16, 16)] = zer
        return 0
    lax.fori_loop(0, _NBUCKET // 16, zi, 0)

    def zc(i, _):
        cvals[pl.ds(i * 16, 16)] = neg
        cidx[pl.ds(i * 16, 16)] = zer
        return 0
    lax.fori_loop(0, _CAP // 16, zc, 0)

    def dma(w, b, sem):
        return pltpu.make_async_copy(
            logits_hbm.at[wid, pl.ds(w * _WIN, _WIN)], wbuf.at[b], sem)

    def scan_window(w, b, gb_in):
        def seg(s, carry):
            gb2, wmx = carry

            def inner(i, ms):
                base = s * (_SEGV * 16) + i * 128
                return tuple(
                    jnp.maximum(ms[u], wbuf[b, pl.ds(base + u * 16, 16)])
                    for u in range(8))
            ms = lax.fori_loop(0, _SEGV // 8, inner, (neg,) * 8)
            m01 = jnp.maximum(ms[0], ms[1])
            m23 = jnp.maximum(ms[2], ms[3])
            m45 = jnp.maximum(ms[4], ms[5])
            m67 = jnp.maximum(ms[6], ms[7])
            m = jnp.maximum(jnp.maximum(m01, m23), jnp.maximum(m45, m67))
            bv = _bucketize(m)
            plsc.addupdate_scatter(hist, [bv], ones16)
            sb = jnp.max(bv)
            segb[w * _NSEG + s] = sb
            return (jnp.maximum(gb2, sb), jnp.maximum(wmx, sb))
        gb_out, wmx = lax.fori_loop(0, _NSEG, seg, (gb_in, jnp.int32(0)))
        wmaxb[w] = wmx
        return gb_out

    dma(0, 0, sem0).start()
    dma(1, 1, sem1).start()

    def pair(g, gb):
        w0 = 2 * g
        dma(w0, 0, sem0).wait()
        gb = scan_window(w0, 0, gb)
        dma(w0 + 2, 0, sem0).start()
        dma(w0 + 1, 1, sem1).wait()
        gb = scan_window(w0 + 1, 1, gb)

        @pl.when(w0 + 3 < _NWIN)
        def _():
            dma(w0 + 3, 1, sem1).start()
        return gb

    gb = lax.fori_loop(0, (_NWIN - 1) // 2, pair, jnp.int32(0))
    dma(_NWIN - 1, 0, sem0).wait()
    bmax = scan_window(_NWIN - 1, 0, gb)

    def cond(c):
        return c[2] < 0

    def wb(c):
        j, cum, bst = c
        v = hist[pl.ds(j * 16, 16)]
        rv = lax.rev(v, (0,))
        cs = jnp.cumsum(rv)
        nhit = jnp.sum((cs + cum >= _K).astype(jnp.int32))
        bst2 = jnp.where(nhit > 0, j * 16 + nhit - 1, bst)
        return (j - 1, cum + jnp.sum(v), bst2)

    _, _, bstar = lax.while_loop(
        cond, wb, (bmax >> 4, jnp.int32(0), jnp.int32(-1)))
    beff = jnp.maximum(bstar - 1, 0)

    def cwin(w, off):
        def do(o):
            pltpu.sync_copy(logits_hbm.at[wid, pl.ds(w * _WIN, _WIN)],
                            wbuf.at[0])
            base = w * _WIN

            def seg_c(s, o2):
                def do2(o3):
                    def inner(i, o4):
                        j = s * _SEGV + i
                        x = wbuf[0, pl.ds(j * 16, 16)]
                        msk = _bucketize(x) >= beff
                        oc = jnp.minimum(o4, _CAP - 16)
                        plsc.store_compressed(cvals.at[pl.ds(oc, 16)], x,
                                              mask=msk)
                        plsc.store_compressed(cidx.at[pl.ds(oc, 16)],
                                              iota16 + (base + j * 16),
                                              mask=msk)
                        return o4 + jnp.sum(msk.astype(jnp.int32))
                    return lax.fori_loop(0, _SEGV, inner, o3)
                return lax.cond(segb[w * _NSEG + s] >= beff, do2,
                                lambda o3: o3, o2)
            return lax.fori_loop(0, _NSEG, seg_c, o)
        return lax.cond(wmaxb[w] >= beff, do, lambda o: o, off)

    lax.fori_loop(0, _NWIN, cwin, jnp.int32(0))

    pltpu.sync_copy(cvals, vals_hbm.at[wid])
    pltpu.sync_copy(cidx, idx_hbm.at[wid])


_select_candidates = pl.kernel(
    _select_body,
    out_type=[jax.ShapeDtypeStruct((_B, _CAP), jnp.float32),
              jax.ShapeDtypeStruct((_B, _CAP), jnp.int32)],
    mesh=_sc_mesh,
    compiler_params=pltpu.CompilerParams(needs_layout_passes=False),
    scratch_types=[
        pltpu.VMEM((_NBUCKET,), jnp.int32),
        pltpu.VMEM((2, _WIN), jnp.float32),
        pltpu.VMEM((_CAP,), jnp.float32),
        pltpu.VMEM((_CAP,), jnp.int32),
        pltpu.SMEM((_NWIN + 1,), jnp.int32),
        pltpu.SMEM((_NWIN * _NSEG,), jnp.int32),
        pltpu.SemaphoreType.DMA,
        pltpu.SemaphoreType.DMA,
    ],
)


def _gather_body(params_hbm, rows_hbm, out_hbm, idxv, rowsv, sem):
    wid = _worker_id()
    pltpu.sync_copy(rows_hbm.at[wid], idxv)
    pltpu.async_copy(params_hbm.at[idxv], rowsv, sem).wait()
    pltpu.sync_copy(rowsv, out_hbm.at[wid])


_gather_rows = pl.kernel(
    _gather_body,
    out_type=jax.ShapeDtypeStruct((_B, _GPAD, _GD), jnp.float32),
    mesh=_sc_mesh,
    compiler_params=pltpu.CompilerParams(
        needs_layout_passes=False, use_tc_tiling_on_sc=False),
    scratch_types=[
        pltpu.VMEM((_GPAD,), jnp.int32),
        pltpu.VMEM((_GPAD, _GD), jnp.float32),
        pltpu.SemaphoreType.DMA,
    ],
)


def kernel(pred_logits, pred_params, target_sizes):
    B, N, C = pred_logits.shape
    flat = pred_logits.reshape(B, N * C)
    cand_vals, cand_idx = _select_candidates(flat)
    p = jax.nn.sigmoid(cand_vals)
    scores, pos = lax.top_k(p, _K)
    flat_idx = jnp.take_along_axis(cand_idx, pos, axis=1)
    labels = flat_idx % C
    rowbase = (jnp.arange(B, dtype=jnp.int32) * N)[:, None]
    rows = flat_idx // C + rowbase
    pad = jnp.broadcast_to(rowbase, (B, _GPAD - _K))
    rows_pad = jnp.concatenate([rows, pad], axis=1).astype(jnp.int32)
    params_pad = jnp.pad(pred_params.reshape(B * N, -1),
                         ((0, 0), (0, _GD - pred_params.shape[2])))
    boxes_g = _gather_rows(params_pad, rows_pad)
    img_h = target_sizes[:, 0].astype(jnp.float32)
    img_w = target_sizes[:, 1].astype(jnp.float32)
    scale_fct = jnp.stack([img_w, img_h] * 9, axis=1)
    boxes = boxes_g[:, :_K, :pred_params.shape[2]] * scale_fct[:, None, :]
    return scores, labels, boxes


# speedup vs baseline: 8.7132x; 1.0313x over previous
import jax
import jax.numpy as jnp
from jax import lax
from jax.experimental import pallas as pl
from jax.experimental.pallas import tpu as pltpu
from jax.experimental.pallas import tpu_sc as plsc

_B = 32
_N = 8192
_C = 91
_NC = _N * _C
_K = 100
_WIN = 28672
_NWIN = _NC // _WIN
_HB = 15
_NBUCKET = 1 << _HB
_SHIFT = 32 - _HB
_CAP = 512
_SEGV = 64
_NSEG = _WIN // (_SEGV * 16)
_NSEGTOT = _NC // (_SEGV * 16)
_GPAD = 128
_GD = 24
_MININT = -(2 ** 31)

_sc_mesh = plsc.VectorSubcoreMesh(core_axis_name="c", subcore_axis_name="s")


def _worker_id():
    return lax.axis_index("s") * 2 + lax.axis_index("c")


def _bucketize(x):
    bits = lax.bitcast_convert_type(x, jnp.int32)
    m = lax.shift_right_arithmetic(bits, 31)
    key_u = bits ^ (m | _MININT)
    return lax.shift_right_logical(key_u, _SHIFT)


def _select_body(logits_hbm, vals_hbm, idx_hbm,
                 hist, wbuf, cbuf, cvals, cidx, segb, sem0, sem1):
    wid = _worker_id()
    zer = jnp.zeros((16,), jnp.int32)
    ones16 = jnp.ones((16,), jnp.int32)
    neg = jnp.full((16,), -jnp.inf, jnp.float32)
    iota16 = lax.iota(jnp.int32, 16)

    def zi(i, _):
        hist[pl.ds(i * 16, 16)] = zer
        return 0
    lax.fori_loop(0, _NBUCKET // 16, zi, 0)

    def zc(i, _):
        cvals[pl.ds(i * 16, 16)] = neg
        cidx[pl.ds(i * 16, 16)] = zer
        return 0
    lax.fori_loop(0, _CAP // 16, zc, 0)

    def dma(w, b, sem):
        return pltpu.make_async_copy(
            logits_hbm.at[wid, pl.ds(w * _WIN, _WIN)], wbuf.at[b], sem)

    def scan_window(w, b, gb_in):
        def seg(s, gb2):
            def inner(i, ms):
                base = s * (_SEGV * 16) + i * 128
                return tuple(
                    jnp.maximum(ms[u], wbuf[b, pl.ds(base + u * 16, 16)])
                    for u in range(8))
            ms = lax.fori_loop(0, _SEGV // 8, inner, (neg,) * 8)
            m01 = jnp.maximum(ms[0], ms[1])
            m23 = jnp.maximum(ms[2], ms[3])
            m45 = jnp.maximum(ms[4], ms[5])
            m67 = jnp.maximum(ms[6], ms[7])
            m = jnp.maximum(jnp.maximum(m01, m23), jnp.maximum(m45, m67))
            bv = _bucketize(m)
            plsc.addupdate_scatter(hist, [bv], ones16)
            sb = jnp.max(bv)
            segb[w * _NSEG + s] = sb
            return jnp.maximum(gb2, sb)
        return lax.fori_loop(0, _NSEG, seg, gb_in)

    dma(0, 0, sem0).start()
    dma(1, 1, sem1).start()

    def pair(g, gb):
        w0 = 2 * g
        dma(w0, 0, sem0).wait()

        @pl.when(w0 + 2 < _NWIN)
        def _():
            dma(w0 + 2, 0, sem0).start()
        gb = scan_window(w0, 0, gb)
        dma(w0 + 1, 1, sem1).wait()

        @pl.when(w0 + 3 < _NWIN)
        def _():
            dma(w0 + 3, 1, sem1).start()
        gb = scan_window(w0 + 1, 1, gb)
        return gb

    bmax = lax.fori_loop(0, _NWIN // 2, pair, jnp.int32(0))

    def cond(c):
        return c[2] < 0

    def wb(c):
        j, cum, bst = c
        v = hist[pl.ds(j * 16, 16)]
        rv = lax.rev(v, (0,))
        cs = jnp.cumsum(rv)
        nhit = jnp.sum((cs + cum >= _K).astype(jnp.int32))
        bst2 = jnp.where(nhit > 0, j * 16 + nhit - 1, bst)
        return (j - 1, cum + jnp.sum(v), bst2)

    _, _, bstar = lax.while_loop(
        cond, wb, (bmax >> 4, jnp.int32(0), jnp.int32(-1)))
    beff = jnp.maximum(bstar - 1, 0)

    def cseg(s, off):
        def do(o):
            pltpu.sync_copy(
                logits_hbm.at[wid, pl.ds(s * (_SEGV * 16), _SEGV * 16)], cbuf)
            base = s * (_SEGV * 16)

            def inner(i, o4):
                x = cbuf[pl.ds(i * 16, 16)]
                msk = _bucketize(x) >= beff
                oc = jnp.minimum(o4, _CAP - 16)
                plsc.store_compressed(cvals.at[pl.ds(oc, 16)], x, mask=msk)
                plsc.store_compressed(cidx.at[pl.ds(oc, 16)],
                                      iota16 + (base + i * 16), mask=msk)
                return o4 + jnp.sum(msk.astype(jnp.int32))
            return lax.fori_loop(0, _SEGV, inner, o)
        return lax.cond(segb[s] >= beff, do, lambda o: o, off)

    lax.fori_loop(0, _NSEGTOT, cseg, jnp.int32(0))

    pltpu.sync_copy(cvals, vals_hbm.at[wid])
    pltpu.sync_copy(cidx, idx_hbm.at[wid])


_select_candidates = pl.kernel(
    _select_body,
    out_type=[jax.ShapeDtypeStruct((_B, _CAP), jnp.float32),
              jax.ShapeDtypeStruct((_B, _CAP), jnp.int32)],
    mesh=_sc_mesh,
    compiler_params=pltpu.CompilerParams(needs_layout_passes=False),
    scratch_types=[
        pltpu.VMEM((_NBUCKET,), jnp.int32),
        pltpu.VMEM((2, _WIN), jnp.float32),
        pltpu.VMEM((_SEGV * 16,), jnp.float32),
        pltpu.VMEM((_CAP,), jnp.float32),
        pltpu.VMEM((_CAP,), jnp.int32),
        pltpu.SMEM((_NSEGTOT,), jnp.int32),
        pltpu.SemaphoreType.DMA,
        pltpu.SemaphoreType.DMA,
    ],
)


def _gather_body(params_hbm, rows_hbm, out_hbm, idxv, rowsv, sem):
    wid = _worker_id()
    pltpu.sync_copy(rows_hbm.at[wid], idxv)
    pltpu.async_copy(params_hbm.at[idxv], rowsv, sem).wait()
    pltpu.sync_copy(rowsv, out_hbm.at[wid])


_gather_rows = pl.kernel(
    _gather_body,
    out_type=jax.ShapeDtypeStruct((_B, _GPAD, _GD), jnp.float32),
    mesh=_sc_mesh,
    compiler_params=pltpu.CompilerParams(
        needs_layout_passes=False, use_tc_tiling_on_sc=False),
    scratch_types=[
        pltpu.VMEM((_GPAD,), jnp.int32),
        pltpu.VMEM((_GPAD, _GD), jnp.float32),
        pltpu.SemaphoreType.DMA,
    ],
)


def kernel(pred_logits, pred_params, target_sizes):
    B, N, C = pred_logits.shape
    flat = pred_logits.reshape(B, N * C)
    cand_vals, cand_idx = _select_candidates(flat)
    p = jax.nn.sigmoid(cand_vals)
    scores, pos = lax.top_k(p, _K)
    flat_idx = jnp.take_along_axis(cand_idx, pos, axis=1)
    labels = flat_idx % C
    rowbase = (jnp.arange(B, dtype=jnp.int32) * N)[:, None]
    rows = flat_idx // C + rowbase
    pad = jnp.broadcast_to(rowbase, (B, _GPAD - _K))
    rows_pad = jnp.concatenate([rows, pad], axis=1).astype(jnp.int32)
    params_pad = jnp.pad(pred_params.reshape(B * N, -1),
                         ((0, 0), (0, _GD - pred_params.shape[2])))
    boxes_g = _gather_rows(params_pad, rows_pad)
    img_h = target_sizes[:, 0].astype(jnp.float32)
    img_w = target_sizes[:, 1].astype(jnp.float32)
    scale_fct = jnp.stack([img_w, img_h] * 9, axis=1)
    boxes = boxes_g[:, :_K, :pred_params.shape[2]] * scale_fct[:, None, :]
    return scores, labels, boxes
